# baseline (speedup 1.0000x reference)
import functools

import jax
import jax.numpy as jnp
from jax.experimental import pallas as pl
from jax.experimental.pallas import tpu as pltpu
from jax.experimental.pallas import tpu_sc as plsc

_SC_NC = 2
_SC_NS = 16
_SC_NW = _SC_NC * _SC_NS


def _dot(a, b, dims):
    return jax.lax.dot_general(a, b, (dims, ((), ())),
                               preferred_element_type=jnp.float32)


def _sc_gather_call(T, idxf):
    NIDX = idxf.shape[0]
    D = T.shape[1]
    per_w = NIDX // _SC_NW
    CH = 64
    n_ch = per_w // CH
    mesh = plsc.VectorSubcoreMesh(core_axis_name="c", subcore_axis_name="s")

    @functools.partial(
        pl.kernel, mesh=mesh,
        out_type=jax.ShapeDtypeStruct((NIDX, D), jnp.float32),
        scratch_types=[
            pltpu.VMEM((CH,), jnp.int32),
            pltpu.VMEM((CH, D), jnp.float32),
            pltpu.SemaphoreType.DMA,
        ],
    )
    def k(table_hbm, idx_hbm, out_hbm, idx_v, rows_v, sem):
        wid = jax.lax.axis_index("s") * _SC_NC + jax.lax.axis_index("c")
        base = wid * per_w

        @pl.loop(0, n_ch)
        def _(i):
            off = base + i * CH
            pltpu.sync_copy(idx_hbm.at[pl.ds(off, CH)], idx_v)
            pltpu.async_copy(table_hbm.at[idx_v], rows_v, sem).wait()
            pltpu.sync_copy(rows_v, out_hbm.at[pl.ds(off, CH)])

    return k(T, idxf)


def _topk_kernel(S, xyz1_ref, xyz2t_ref, idx_ref, w_ref):
    x = xyz1_ref[0]
    q = xyz2t_ref[0]
    nb = x.shape[0]
    cross = jax.lax.dot_general(x.astype(jnp.bfloat16), q.astype(jnp.bfloat16),
                                (((1,), (0,)), ((), ())),
                                preferred_element_type=jnp.float32)
    xs = (x[:, 0:1] * x[:, 0:1] + x[:, 1:2] * x[:, 1:2]) + x[:, 2:3] * x[:, 2:3]
    qs = (q[0:1, :] * q[0:1, :] + q[1:2, :] * q[1:2, :]) + q[2:3, :] * q[2:3, :]
    d = (-2.0 * cross + xs) + qs
    iota = jax.lax.broadcasted_iota(jnp.int32, (nb, S), 1)
    idxs, vals = [], []
    for k in range(3):
        mval = jnp.min(d, axis=1, keepdims=True)
        am = jnp.min(jnp.where(d == mval, iota, S), axis=1, keepdims=True)
        idxs.append(am)
        vals.append(mval)
        if k < 2:
            d = jnp.where(iota == am, jnp.float32(jnp.inf), d)
    d3 = jnp.concatenate(vals, axis=1)
    recip = 1.0 / (d3 + 1e-8)
    w = recip / jnp.sum(recip, axis=1, keepdims=True)
    b = pl.program_id(0)
    gidx = jnp.concatenate(idxs, axis=1) + b * S
    idx_ref[...] = gidx.T
    w_ref[...] = w


def _table_kernel(p2_ref, Wc2_ref, T_ref):
    T_ref[...] = _dot(p2_ref[0], Wc2_ref[...], (((0,), (1,))))


def _fuse_kernel(g0_ref, g1_ref, g2_ref, w_ref, p1_ref, Wc1_ref, s1_ref,
                 st_ref):
    b = pl.program_id(0)
    j = pl.program_id(1)
    w = w_ref[...]
    s1 = (g0_ref[...] * w[:, 0:1] + g1_ref[...] * w[:, 1:2]
          + g2_ref[...] * w[:, 2:3])
    s1 = s1 + _dot(p1_ref[0].astype(jnp.bfloat16),
                   Wc1_ref[...].astype(jnp.bfloat16), (((0,), (1,))))
    s1_ref[...] = s1

    @pl.when((b == 0) & (j == 0))
    def _():
        st_ref[...] = jnp.zeros_like(st_ref)
    st_ref[0:1, :] += jnp.sum(s1, axis=0, keepdims=True)
    st_ref[1:2, :] += jnp.sum(s1 * s1, axis=0, keepdims=True)


def _mid_kernel(store_x, s_ref, sc_ref, sh_ref, W_ref, *out_refs):
    if store_x:
        x_ref, s2_ref, st_ref = out_refs
    else:
        s2_ref, st_ref = out_refs
    x = jnp.maximum(s_ref[...] * sc_ref[...] + sh_ref[...], 0.0)
    s2 = _dot(x.astype(jnp.bfloat16), W_ref[...].astype(jnp.bfloat16),
              (((1,), (1,))))
    if store_x:
        x_ref[...] = x
    s2_ref[...] = s2

    @pl.when(pl.program_id(0) == 0)
    def _():
        st_ref[...] = jnp.zeros_like(st_ref)
    st_ref[0:1, :] += jnp.sum(s2, axis=0, keepdims=True)
    st_ref[1:2, :] += jnp.sum(s2 * s2, axis=0, keepdims=True)


def _final_kernel(s3_ref, x_ref, sc_ref, sh_ref, o_ref):
    y = s3_ref[...] * sc_ref[...] + sh_ref[...] + x_ref[...]
    o_ref[0] = jnp.maximum(y, 0.0).T


def _stats_to_scale_shift(st, nt, g, be, eps):
    mean = st[0] / nt
    var = st[1] / nt - mean * mean
    scale = g / jnp.sqrt(var + eps)
    shift = be - mean * scale
    return scale[None, :], shift[None, :]


def kernel(xyz1, xyz2, points1, points2, W_fuse, b_fuse, g_fuse, be_fuse,
           W1, b1, g1, be1, W2, b2, g2, be2):
    B, N, _ = xyz1.shape
    S = xyz2.shape[1]
    C1 = points1.shape[1]
    C2 = points2.shape[1]
    CO = W_fuse.shape[0]
    NT = B * N
    nb = 512
    NB = N // nb
    f32 = jnp.float32

    xyz2t = jnp.transpose(xyz2, (0, 2, 1))
    Wc1 = W_fuse[:, :C1]
    Wc2 = W_fuse[:, C1:]

    idx, w = pl.pallas_call(
        functools.partial(_topk_kernel, S),
        grid=(B, NB),
        in_specs=[
            pl.BlockSpec((1, nb, 3), lambda b, j: (b, j, 0)),
            pl.BlockSpec((1, 3, S), lambda b, j: (b, 0, 0)),
        ],
        out_specs=[
            pl.BlockSpec((3, nb), lambda b, j: (0, b * NB + j)),
            pl.BlockSpec((nb, 3), lambda b, j: (b * NB + j, 0)),
        ],
        out_shape=[
            jax.ShapeDtypeStruct((3, NT), jnp.int32),
            jax.ShapeDtypeStruct((NT, 3), f32),
        ],
    )(xyz1, xyz2t)

    T = pl.pallas_call(
        _table_kernel,
        grid=(B,),
        in_specs=[
            pl.BlockSpec((1, C2, S), lambda b: (b, 0, 0)),
            pl.BlockSpec((CO, C2), lambda b: (0, 0)),
        ],
        out_specs=pl.BlockSpec((S, CO), lambda b: (b, 0)),
        out_shape=jax.ShapeDtypeStruct((B * S, CO), f32),
    )(points2, Wc2)

    G = _sc_gather_call(T, jnp.reshape(idx, (3 * NT,)))

    NBT = NT // nb
    s1, st1 = pl.pallas_call(
        _fuse_kernel,
        grid=(B, NB),
        in_specs=[
            pl.BlockSpec((nb, CO), lambda b, j: (0 * NBT + b * NB + j, 0)),
            pl.BlockSpec((nb, CO), lambda b, j: (1 * NBT + b * NB + j, 0)),
            pl.BlockSpec((nb, CO), lambda b, j: (2 * NBT + b * NB + j, 0)),
            pl.BlockSpec((nb, 3), lambda b, j: (b * NB + j, 0)),
            pl.BlockSpec((1, C1, nb), lambda b, j: (b, 0, j)),
            pl.BlockSpec((CO, C1), lambda b, j: (0, 0)),
        ],
        out_specs=[
            pl.BlockSpec((nb, CO), lambda b, j: (b * NB + j, 0)),
            pl.BlockSpec((8, CO), lambda b, j: (0, 0)),
        ],
        out_shape=[
            jax.ShapeDtypeStruct((NT, CO), f32),
            jax.ShapeDtypeStruct((8, CO), f32),
        ],
    )(G, G, G, w, points1, Wc1)

    sc1, sh1 = _stats_to_scale_shift(st1, NT, g_fuse, be_fuse, 1e-5)

    x, s2, st2 = pl.pallas_call(
        functools.partial(_mid_kernel, True),
        grid=(NBT,),
        in_specs=[
            pl.BlockSpec((nb, CO), lambda i: (i, 0)),
            pl.BlockSpec((1, CO), lambda i: (0, 0)),
            pl.BlockSpec((1, CO), lambda i: (0, 0)),
            pl.BlockSpec((CO, CO), lambda i: (0, 0)),
        ],
        out_specs=[
            pl.BlockSpec((nb, CO), lambda i: (i, 0)),
            pl.BlockSpec((nb, CO), lambda i: (i, 0)),
            pl.BlockSpec((8, CO), lambda i: (0, 0)),
        ],
        out_shape=[
            jax.ShapeDtypeStruct((NT, CO), f32),
            jax.ShapeDtypeStruct((NT, CO), f32),
            jax.ShapeDtypeStruct((8, CO), f32),
        ],
    )(s1, sc1, sh1, W1)

    sc2, sh2 = _stats_to_scale_shift(st2, NT, g1, be1, 1e-5)

    s3, st3 = pl.pallas_call(
        functools.partial(_mid_kernel, False),
        grid=(NBT,),
        in_specs=[
            pl.BlockSpec((nb, CO), lambda i: (i, 0)),
            pl.BlockSpec((1, CO), lambda i: (0, 0)),
            pl.BlockSpec((1, CO), lambda i: (0, 0)),
            pl.BlockSpec((CO, CO), lambda i: (0, 0)),
        ],
        out_specs=[
            pl.BlockSpec((nb, CO), lambda i: (i, 0)),
            pl.BlockSpec((8, CO), lambda i: (0, 0)),
        ],
        out_shape=[
            jax.ShapeDtypeStruct((NT, CO), f32),
            jax.ShapeDtypeStruct((8, CO), f32),
        ],
    )(s2, sc2, sh2, W2)

    sc3, sh3 = _stats_to_scale_shift(st3, NT, g2, be2, 1e-5)

    out = pl.pallas_call(
        _final_kernel,
        grid=(B, NB),
        in_specs=[
            pl.BlockSpec((nb, CO), lambda b, j: (b * NB + j, 0)),
            pl.BlockSpec((nb, CO), lambda b, j: (b * NB + j, 0)),
            pl.BlockSpec((1, CO), lambda b, j: (0, 0)),
            pl.BlockSpec((1, CO), lambda b, j: (0, 0)),
        ],
        out_specs=pl.BlockSpec((1, CO, nb), lambda b, j: (b, 0, j)),
        out_shape=jax.ShapeDtypeStruct((B, CO, N), f32),
    )(s3, x, sc3, sh3)

    return out


# speedup vs baseline: 1.0832x; 1.0832x over previous
import functools

import jax
import jax.numpy as jnp
from jax.experimental import pallas as pl
from jax.experimental.pallas import tpu as pltpu
from jax.experimental.pallas import tpu_sc as plsc

_SC_NC = 2
_SC_NS = 16
_SC_NW = _SC_NC * _SC_NS


def _dot(a, b, dims):
    return jax.lax.dot_general(a, b, (dims, ((), ())),
                               preferred_element_type=jnp.float32)


def _sc_gather_call(T, idxf):
    NIDX = idxf.shape[0]
    D = T.shape[1]
    per_w = NIDX // _SC_NW
    CH = 96
    n_pair = per_w // (2 * CH)
    mesh = plsc.VectorSubcoreMesh(core_axis_name="c", subcore_axis_name="s")

    @functools.partial(
        pl.kernel, mesh=mesh,
        out_type=jax.ShapeDtypeStruct((NIDX, D), jnp.float32),
        scratch_types=[
            pltpu.VMEM((per_w,), jnp.int32),
            pltpu.VMEM((CH, D), jnp.float32),
            pltpu.VMEM((CH, D), jnp.float32),
            pltpu.SemaphoreType.DMA,
            pltpu.SemaphoreType.DMA,
            pltpu.SemaphoreType.DMA,
            pltpu.SemaphoreType.DMA,
        ],
    )
    def k(table_hbm, idx_hbm, out_hbm, idx_v, rows0, rows1,
          g0, g1, o0, o1):
        wid = jax.lax.axis_index("s") * _SC_NC + jax.lax.axis_index("c")
        base = wid * per_w
        pltpu.sync_copy(idx_hbm.at[pl.ds(base, per_w)], idx_v)

        def gather(c, buf, sem):
            return pltpu.make_async_copy(
                table_hbm.at[idx_v.at[pl.ds(c * CH, CH)]], buf, sem)

        def store(c, buf, sem):
            return pltpu.make_async_copy(
                buf, out_hbm.at[pl.ds(base + c * CH, CH)], sem)

        gather(0, rows0, g0).start()

        @pl.loop(0, n_pair)
        def _(p):
            a = 2 * p
            gather(a, rows0, g0).wait()
            gather(a + 1, rows1, g1).start()
            store(a, rows0, o0).start()
            gather(a + 1, rows1, g1).wait()
            store(a, rows0, o0).wait()

            @pl.when(p + 1 < n_pair)
            def _():
                gather(a + 2, rows0, g0).start()
            store(a + 1, rows1, o1).start()
            store(a + 1, rows1, o1).wait()

    return k(T, idxf)


def _topk_kernel(S, xyz1_ref, xyz2t_ref, idx_ref, w_ref):
    x = xyz1_ref[0]
    q = xyz2t_ref[0]
    nb = x.shape[0]
    cross = jax.lax.dot_general(x.astype(jnp.bfloat16), q.astype(jnp.bfloat16),
                                (((1,), (0,)), ((), ())),
                                preferred_element_type=jnp.float32)
    xs = (x[:, 0:1] * x[:, 0:1] + x[:, 1:2] * x[:, 1:2]) + x[:, 2:3] * x[:, 2:3]
    qs = (q[0:1, :] * q[0:1, :] + q[1:2, :] * q[1:2, :]) + q[2:3, :] * q[2:3, :]
    d = (-2.0 * cross + xs) + qs
    iota = jax.lax.broadcasted_iota(jnp.int32, (nb, S), 1)
    idxs, vals = [], []
    for k in range(3):
        mval = jnp.min(d, axis=1, keepdims=True)
        am = jnp.min(jnp.where(d == mval, iota, S), axis=1, keepdims=True)
        idxs.append(am)
        vals.append(mval)
        if k < 2:
            d = jnp.where(iota == am, jnp.float32(jnp.inf), d)
    d3 = jnp.concatenate(vals, axis=1)
    recip = 1.0 / (d3 + 1e-8)
    w = recip / jnp.sum(recip, axis=1, keepdims=True)
    b = pl.program_id(0)
    gidx = jnp.concatenate(idxs, axis=1) + b * S
    idx_ref[...] = gidx.T
    w_ref[...] = w


def _table_kernel(p2_ref, Wc2_ref, T_ref):
    T_ref[...] = _dot(p2_ref[0], Wc2_ref[...], (((0,), (1,))))


def _fuse_kernel(g0_ref, g1_ref, g2_ref, w_ref, p1_ref, Wc1_ref, s1_ref,
                 st_ref):
    b = pl.program_id(0)
    j = pl.program_id(1)
    w = w_ref[...]
    s1 = (g0_ref[...] * w[:, 0:1] + g1_ref[...] * w[:, 1:2]
          + g2_ref[...] * w[:, 2:3])
    s1 = s1 + _dot(p1_ref[0].astype(jnp.bfloat16),
                   Wc1_ref[...].astype(jnp.bfloat16), (((0,), (1,))))
    s1_ref[...] = s1

    @pl.when((b == 0) & (j == 0))
    def _():
        st_ref[...] = jnp.zeros_like(st_ref)
    st_ref[0:1, :] += jnp.sum(s1, axis=0, keepdims=True)
    st_ref[1:2, :] += jnp.sum(s1 * s1, axis=0, keepdims=True)


def _mid_kernel(store_x, s_ref, sc_ref, sh_ref, W_ref, *out_refs):
    if store_x:
        x_ref, s2_ref, st_ref = out_refs
    else:
        s2_ref, st_ref = out_refs
    x = jnp.maximum(s_ref[...] * sc_ref[...] + sh_ref[...], 0.0)
    s2 = _dot(x.astype(jnp.bfloat16), W_ref[...].astype(jnp.bfloat16),
              (((1,), (1,))))
    if store_x:
        x_ref[...] = x
    s2_ref[...] = s2

    @pl.when(pl.program_id(0) == 0)
    def _():
        st_ref[...] = jnp.zeros_like(st_ref)
    st_ref[0:1, :] += jnp.sum(s2, axis=0, keepdims=True)
    st_ref[1:2, :] += jnp.sum(s2 * s2, axis=0, keepdims=True)


def _final_kernel(s3_ref, x_ref, sc_ref, sh_ref, o_ref):
    y = s3_ref[...] * sc_ref[...] + sh_ref[...] + x_ref[...]
    o_ref[0] = jnp.maximum(y, 0.0).T


def _stats_to_scale_shift(st, nt, g, be, eps):
    mean = st[0] / nt
    var = st[1] / nt - mean * mean
    scale = g / jnp.sqrt(var + eps)
    shift = be - mean * scale
    return scale[None, :], shift[None, :]


def kernel(xyz1, xyz2, points1, points2, W_fuse, b_fuse, g_fuse, be_fuse,
           W1, b1, g1, be1, W2, b2, g2, be2):
    B, N, _ = xyz1.shape
    S = xyz2.shape[1]
    C1 = points1.shape[1]
    C2 = points2.shape[1]
    CO = W_fuse.shape[0]
    NT = B * N
    nb = 512
    NB = N // nb
    f32 = jnp.float32

    xyz2t = jnp.transpose(xyz2, (0, 2, 1))
    Wc1 = W_fuse[:, :C1]
    Wc2 = W_fuse[:, C1:]

    idx, w = pl.pallas_call(
        functools.partial(_topk_kernel, S),
        grid=(B, NB),
        in_specs=[
            pl.BlockSpec((1, nb, 3), lambda b, j: (b, j, 0)),
            pl.BlockSpec((1, 3, S), lambda b, j: (b, 0, 0)),
        ],
        out_specs=[
            pl.BlockSpec((3, nb), lambda b, j: (0, b * NB + j)),
            pl.BlockSpec((nb, 3), lambda b, j: (b * NB + j, 0)),
        ],
        out_shape=[
            jax.ShapeDtypeStruct((3, NT), jnp.int32),
            jax.ShapeDtypeStruct((NT, 3), f32),
        ],
    )(xyz1, xyz2t)

    T = pl.pallas_call(
        _table_kernel,
        grid=(B,),
        in_specs=[
            pl.BlockSpec((1, C2, S), lambda b: (b, 0, 0)),
            pl.BlockSpec((CO, C2), lambda b: (0, 0)),
        ],
        out_specs=pl.BlockSpec((S, CO), lambda b: (b, 0)),
        out_shape=jax.ShapeDtypeStruct((B * S, CO), f32),
    )(points2, Wc2)

    G = _sc_gather_call(T, jnp.reshape(idx, (3 * NT,)))

    NBT = NT // nb
    s1, st1 = pl.pallas_call(
        _fuse_kernel,
        grid=(B, NB),
        in_specs=[
            pl.BlockSpec((nb, CO), lambda b, j: (0 * NBT + b * NB + j, 0)),
            pl.BlockSpec((nb, CO), lambda b, j: (1 * NBT + b * NB + j, 0)),
            pl.BlockSpec((nb, CO), lambda b, j: (2 * NBT + b * NB + j, 0)),
            pl.BlockSpec((nb, 3), lambda b, j: (b * NB + j, 0)),
            pl.BlockSpec((1, C1, nb), lambda b, j: (b, 0, j)),
            pl.BlockSpec((CO, C1), lambda b, j: (0, 0)),
        ],
        out_specs=[
            pl.BlockSpec((nb, CO), lambda b, j: (b * NB + j, 0)),
            pl.BlockSpec((8, CO), lambda b, j: (0, 0)),
        ],
        out_shape=[
            jax.ShapeDtypeStruct((NT, CO), f32),
            jax.ShapeDtypeStruct((8, CO), f32),
        ],
    )(G, G, G, w, points1, Wc1)

    sc1, sh1 = _stats_to_scale_shift(st1, NT, g_fuse, be_fuse, 1e-5)

    x, s2, st2 = pl.pallas_call(
        functools.partial(_mid_kernel, True),
        grid=(NBT,),
        in_specs=[
            pl.BlockSpec((nb, CO), lambda i: (i, 0)),
            pl.BlockSpec((1, CO), lambda i: (0, 0)),
            pl.BlockSpec((1, CO), lambda i: (0, 0)),
            pl.BlockSpec((CO, CO), lambda i: (0, 0)),
        ],
        out_specs=[
            pl.BlockSpec((nb, CO), lambda i: (i, 0)),
            pl.BlockSpec((nb, CO), lambda i: (i, 0)),
            pl.BlockSpec((8, CO), lambda i: (0, 0)),
        ],
        out_shape=[
            jax.ShapeDtypeStruct((NT, CO), f32),
            jax.ShapeDtypeStruct((NT, CO), f32),
            jax.ShapeDtypeStruct((8, CO), f32),
        ],
    )(s1, sc1, sh1, W1)

    sc2, sh2 = _stats_to_scale_shift(st2, NT, g1, be1, 1e-5)

    s3, st3 = pl.pallas_call(
        functools.partial(_mid_kernel, False),
        grid=(NBT,),
        in_specs=[
            pl.BlockSpec((nb, CO), lambda i: (i, 0)),
            pl.BlockSpec((1, CO), lambda i: (0, 0)),
            pl.BlockSpec((1, CO), lambda i: (0, 0)),
            pl.BlockSpec((CO, CO), lambda i: (0, 0)),
        ],
        out_specs=[
            pl.BlockSpec((nb, CO), lambda i: (i, 0)),
            pl.BlockSpec((8, CO), lambda i: (0, 0)),
        ],
        out_shape=[
            jax.ShapeDtypeStruct((NT, CO), f32),
            jax.ShapeDtypeStruct((8, CO), f32),
        ],
    )(s2, sc2, sh2, W2)

    sc3, sh3 = _stats_to_scale_shift(st3, NT, g2, be2, 1e-5)

    out = pl.pallas_call(
        _final_kernel,
        grid=(B, NB),
        in_specs=[
            pl.BlockSpec((nb, CO), lambda b, j: (b * NB + j, 0)),
            pl.BlockSpec((nb, CO), lambda b, j: (b * NB + j, 0)),
            pl.BlockSpec((1, CO), lambda b, j: (0, 0)),
            pl.BlockSpec((1, CO), lambda b, j: (0, 0)),
        ],
        out_specs=pl.BlockSpec((1, CO, nb), lambda b, j: (b, 0, j)),
        out_shape=jax.ShapeDtypeStruct((B, CO, N), f32),
    )(s3, x, sc3, sh3)

    return out


# speedup vs baseline: 1.4629x; 1.3505x over previous
import functools

import jax
import jax.numpy as jnp
from jax.experimental import pallas as pl
from jax.experimental.pallas import tpu as pltpu
from jax.experimental.pallas import tpu_sc as plsc

_SC_NC = 2
_SC_NS = 16
_SC_NW = _SC_NC * _SC_NS


def _dot(a, b, dims):
    return jax.lax.dot_general(a, b, (dims, ((), ())),
                               preferred_element_type=jnp.float32)


def _sc_gather_call(T, idxf):
    NIDX = idxf.shape[0]
    D = T.shape[1]
    per_w = NIDX // _SC_NW
    CH = 96
    n_pair = per_w // (2 * CH)
    mesh = plsc.VectorSubcoreMesh(core_axis_name="c", subcore_axis_name="s")

    @functools.partial(
        pl.kernel, mesh=mesh,
        out_type=jax.ShapeDtypeStruct((NIDX, D), jnp.float32),
        scratch_types=[
            pltpu.VMEM((per_w,), jnp.int32),
            pltpu.VMEM((CH, D), jnp.float32),
            pltpu.VMEM((CH, D), jnp.float32),
            pltpu.SemaphoreType.DMA,
            pltpu.SemaphoreType.DMA,
            pltpu.SemaphoreType.DMA,
            pltpu.SemaphoreType.DMA,
        ],
    )
    def k(table_hbm, idx_hbm, out_hbm, idx_v, rows0, rows1,
          g0, g1, o0, o1):
        wid = jax.lax.axis_index("s") * _SC_NC + jax.lax.axis_index("c")
        base = wid * per_w
        pltpu.sync_copy(idx_hbm.at[pl.ds(base, per_w)], idx_v)

        def gather(c, buf, sem):
            return pltpu.make_async_copy(
                table_hbm.at[idx_v.at[pl.ds(c * CH, CH)]], buf, sem)

        def store(c, buf, sem):
            return pltpu.make_async_copy(
                buf, out_hbm.at[pl.ds(base + c * CH, CH)], sem)

        gather(0, rows0, g0).start()

        @pl.loop(0, n_pair)
        def _(p):
            a = 2 * p
            gather(a, rows0, g0).wait()
            gather(a + 1, rows1, g1).start()
            store(a, rows0, o0).start()
            gather(a + 1, rows1, g1).wait()
            store(a, rows0, o0).wait()

            @pl.when(p + 1 < n_pair)
            def _():
                gather(a + 2, rows0, g0).start()
            store(a + 1, rows1, o1).start()
            store(a + 1, rows1, o1).wait()

    return k(T, idxf)


def _topk_kernel(S, mode, xyz1_ref, xyz2t_ref, idx_ref, w_ref):
    x = xyz1_ref[0]
    q = xyz2t_ref[0]
    nb = x.shape[0]
    cross = jax.lax.dot_general(x.astype(jnp.bfloat16), q.astype(jnp.bfloat16),
                                (((1,), (0,)), ((), ())),
                                preferred_element_type=jnp.float32)
    xs = (x[:, 0:1] * x[:, 0:1] + x[:, 1:2] * x[:, 1:2]) + x[:, 2:3] * x[:, 2:3]
    qs = (q[0:1, :] * q[0:1, :] + q[1:2, :] * q[1:2, :]) + q[2:3, :] * q[2:3, :]
    d = (-2.0 * cross + xs) + qs
    iota = jax.lax.broadcasted_iota(jnp.int32, (nb, S), 1)
    idxs, vals = [], []
    for k in range(3):
        mval = jnp.min(d, axis=1, keepdims=True)
        am = jnp.min(jnp.where(d == mval, iota, S), axis=1, keepdims=True)
        idxs.append(am)
        vals.append(mval)
        if k < 2:
            d = jnp.where(iota == am, jnp.float32(jnp.inf), d)
    d3 = jnp.concatenate(vals, axis=1)
    recip = 1.0 / (d3 + 1e-8)
    w = recip / jnp.sum(recip, axis=1, keepdims=True)
    idx = jnp.concatenate(idxs, axis=1)
    if mode == "sc":
        idx_ref[...] = (idx + pl.program_id(0) * S).T
    else:
        idx_ref[...] = idx
    w_ref[...] = w


def _table_kernel(p2_ref, Wc2_ref, T_ref):
    T_ref[...] = _dot(p2_ref[0], Wc2_ref[...], (((0,), (1,))))


def _fuse_kernel(S, Bh, g0_ref, g1_ref, g2_ref, wa_ref, idxb_ref, wb_ref,
                 p1_ref, T_ref, Wc1_ref, s1_ref, st_ref):
    b = pl.program_id(0)
    j = pl.program_id(1)
    p1dot = _dot(p1_ref[0].astype(jnp.bfloat16),
                 Wc1_ref[...].astype(jnp.bfloat16), (((0,), (1,))))
    nb = p1dot.shape[0]

    @pl.when(b < Bh)
    def _():
        wa = wa_ref[...]
        s1 = (g0_ref[...] * wa[:, 0:1] + g1_ref[...] * wa[:, 1:2]
              + g2_ref[...] * wa[:, 2:3]) + p1dot
        s1_ref[...] = s1.astype(s1_ref.dtype)

    @pl.when(b >= Bh)
    def _():
        idx = idxb_ref[...]
        wb = wb_ref[...]
        iota = jax.lax.broadcasted_iota(jnp.int32, (nb, S), 1)
        oh = jnp.zeros((nb, S), jnp.float32)
        for k in range(3):
            oh = oh + jnp.where(iota == idx[:, k:k + 1], wb[:, k:k + 1], 0.0)
        s1 = _dot(oh, T_ref[...], (((1,), (0,)))) + p1dot
        s1_ref[...] = s1.astype(s1_ref.dtype)

    @pl.when((b == 0) & (j == 0))
    def _():
        st_ref[...] = jnp.zeros_like(st_ref)
    sv = s1_ref[...].astype(jnp.float32)
    st_ref[0:1, :] += jnp.sum(sv, axis=0, keepdims=True)
    st_ref[1:2, :] += jnp.sum(sv * sv, axis=0, keepdims=True)


def _mid_kernel(store_x, s_ref, sc_ref, sh_ref, W_ref, *out_refs):
    if store_x:
        x_ref, s2_ref, st_ref = out_refs
    else:
        s2_ref, st_ref = out_refs
    x = jnp.maximum(s_ref[...].astype(jnp.float32) * sc_ref[...]
                    + sh_ref[...], 0.0)
    xb = x.astype(jnp.bfloat16)
    s2 = _dot(xb, W_ref[...].astype(jnp.bfloat16), (((1,), (1,))))
    if store_x:
        x_ref[...] = xb
    s2b = s2.astype(jnp.bfloat16)
    s2_ref[...] = s2b

    @pl.when(pl.program_id(0) == 0)
    def _():
        st_ref[...] = jnp.zeros_like(st_ref)
    s2f = s2b.astype(jnp.float32)
    st_ref[0:1, :] += jnp.sum(s2f, axis=0, keepdims=True)
    st_ref[1:2, :] += jnp.sum(s2f * s2f, axis=0, keepdims=True)


def _final_kernel(s3_ref, x_ref, sc_ref, sh_ref, o_ref):
    y = (s3_ref[...].astype(jnp.float32) * sc_ref[...] + sh_ref[...]
         + x_ref[...].astype(jnp.float32))
    o_ref[0] = jnp.maximum(y, 0.0).T


def _stats_to_scale_shift(st, nt, g, be, eps):
    mean = st[0] / nt
    var = st[1] / nt - mean * mean
    scale = g / jnp.sqrt(var + eps)
    shift = be - mean * scale
    return scale[None, :], shift[None, :]


def kernel(xyz1, xyz2, points1, points2, W_fuse, b_fuse, g_fuse, be_fuse,
           W1, b1, g1, be1, W2, b2, g2, be2):
    B, N, _ = xyz1.shape
    S = xyz2.shape[1]
    C1 = points1.shape[1]
    C2 = points2.shape[1]
    CO = W_fuse.shape[0]
    NT = B * N
    nb = 512
    NB = N // nb
    f32 = jnp.float32

    xyz2t = jnp.transpose(xyz2, (0, 2, 1))
    Wc1 = W_fuse[:, :C1]
    Wc2 = W_fuse[:, C1:]
    bf16 = jnp.bfloat16

    Bh = B // 2
    NTh = Bh * N
    NBh = NTh // nb

    T = pl.pallas_call(
        _table_kernel,
        grid=(B,),
        in_specs=[
            pl.BlockSpec((1, C2, S), lambda b: (b, 0, 0)),
            pl.BlockSpec((CO, C2), lambda b: (0, 0)),
        ],
        out_specs=pl.BlockSpec((S, CO), lambda b: (b, 0)),
        out_shape=jax.ShapeDtypeStruct((B * S, CO), f32),
    )(points2, Wc2)

    idxA, wA = pl.pallas_call(
        functools.partial(_topk_kernel, S, "sc"),
        grid=(Bh, NB),
        in_specs=[
            pl.BlockSpec((1, nb, 3), lambda b, j: (b, j, 0)),
            pl.BlockSpec((1, 3, S), lambda b, j: (b, 0, 0)),
        ],
        out_specs=[
            pl.BlockSpec((3, nb), lambda b, j: (0, b * NB + j)),
            pl.BlockSpec((nb, 3), lambda b, j: (b * NB + j, 0)),
        ],
        out_shape=[
            jax.ShapeDtypeStruct((3, NTh), jnp.int32),
            jax.ShapeDtypeStruct((NTh, 3), f32),
        ],
    )(xyz1, xyz2t)

    G = _sc_gather_call(T, jnp.reshape(idxA, (3 * NTh,)))

    idxB, wB = pl.pallas_call(
        functools.partial(_topk_kernel, S, "oh"),
        grid=(Bh, NB),
        in_specs=[
            pl.BlockSpec((1, nb, 3), lambda b, j: (b + Bh, j, 0)),
            pl.BlockSpec((1, 3, S), lambda b, j: (b + Bh, 0, 0)),
        ],
        out_specs=[
            pl.BlockSpec((nb, 3), lambda b, j: (b * NB + j, 0)),
            pl.BlockSpec((nb, 3), lambda b, j: (b * NB + j, 0)),
        ],
        out_shape=[
            jax.ShapeDtypeStruct((NTh, 3), jnp.int32),
            jax.ShapeDtypeStruct((NTh, 3), f32),
        ],
    )(xyz1, xyz2t)

    NBT = NT // nb
    s1, st1 = pl.pallas_call(
        functools.partial(_fuse_kernel, S, Bh),
        grid=(B, NB),
        in_specs=[
            pl.BlockSpec(
                (nb, CO),
                lambda b, j: (0 * NBh + jnp.minimum(b * NB + j, NBh - 1), 0)),
            pl.BlockSpec(
                (nb, CO),
                lambda b, j: (1 * NBh + jnp.minimum(b * NB + j, NBh - 1), 0)),
            pl.BlockSpec(
                (nb, CO),
                lambda b, j: (2 * NBh + jnp.minimum(b * NB + j, NBh - 1), 0)),
            pl.BlockSpec(
                (nb, 3), lambda b, j: (jnp.minimum(b * NB + j, NBh - 1), 0)),
            pl.BlockSpec(
                (nb, 3),
                lambda b, j: (jnp.maximum(b * NB + j - NBh, 0), 0)),
            pl.BlockSpec(
                (nb, 3),
                lambda b, j: (jnp.maximum(b * NB + j - NBh, 0), 0)),
            pl.BlockSpec((1, C1, nb), lambda b, j: (b, 0, j)),
            pl.BlockSpec((S, CO), lambda b, j: (jnp.maximum(b, Bh), 0)),
            pl.BlockSpec((CO, C1), lambda b, j: (0, 0)),
        ],
        out_specs=[
            pl.BlockSpec((nb, CO), lambda b, j: (b * NB + j, 0)),
            pl.BlockSpec((8, CO), lambda b, j: (0, 0)),
        ],
        out_shape=[
            jax.ShapeDtypeStruct((NT, CO), bf16),
            jax.ShapeDtypeStruct((8, CO), f32),
        ],
    )(G, G, G, wA, idxB, wB, points1, T, Wc1)

    sc1, sh1 = _stats_to_scale_shift(st1, NT, g_fuse, be_fuse, 1e-5)

    x, s2, st2 = pl.pallas_call(
        functools.partial(_mid_kernel, True),
        grid=(NBT,),
        in_specs=[
            pl.BlockSpec((nb, CO), lambda i: (i, 0)),
            pl.BlockSpec((1, CO), lambda i: (0, 0)),
            pl.BlockSpec((1, CO), lambda i: (0, 0)),
            pl.BlockSpec((CO, CO), lambda i: (0, 0)),
        ],
        out_specs=[
            pl.BlockSpec((nb, CO), lambda i: (i, 0)),
            pl.BlockSpec((nb, CO), lambda i: (i, 0)),
            pl.BlockSpec((8, CO), lambda i: (0, 0)),
        ],
        out_shape=[
            jax.ShapeDtypeStruct((NT, CO), bf16),
            jax.ShapeDtypeStruct((NT, CO), bf16),
            jax.ShapeDtypeStruct((8, CO), f32),
        ],
    )(s1, sc1, sh1, W1)

    sc2, sh2 = _stats_to_scale_shift(st2, NT, g1, be1, 1e-5)

    s3, st3 = pl.pallas_call(
        functools.partial(_mid_kernel, False),
        grid=(NBT,),
        in_specs=[
            pl.BlockSpec((nb, CO), lambda i: (i, 0)),
            pl.BlockSpec((1, CO), lambda i: (0, 0)),
            pl.BlockSpec((1, CO), lambda i: (0, 0)),
            pl.BlockSpec((CO, CO), lambda i: (0, 0)),
        ],
        out_specs=[
            pl.BlockSpec((nb, CO), lambda i: (i, 0)),
            pl.BlockSpec((8, CO), lambda i: (0, 0)),
        ],
        out_shape=[
            jax.ShapeDtypeStruct((NT, CO), bf16),
            jax.ShapeDtypeStruct((8, CO), f32),
        ],
    )(s2, sc2, sh2, W2)

    sc3, sh3 = _stats_to_scale_shift(st3, NT, g2, be2, 1e-5)

    out = pl.pallas_call(
        _final_kernel,
        grid=(B, NB),
        in_specs=[
            pl.BlockSpec((nb, CO), lambda b, j: (b * NB + j, 0)),
            pl.BlockSpec((nb, CO), lambda b, j: (b * NB + j, 0)),
            pl.BlockSpec((1, CO), lambda b, j: (0, 0)),
            pl.BlockSpec((1, CO), lambda b, j: (0, 0)),
        ],
        out_specs=pl.BlockSpec((1, CO, nb), lambda b, j: (b, 0, j)),
        out_shape=jax.ShapeDtypeStruct((B, CO, N), f32),
    )(s3, x, sc3, sh3)

    return out


# speedup vs baseline: 1.7445x; 1.1925x over previous
import functools

import jax
import jax.numpy as jnp
from jax.experimental import pallas as pl
from jax.experimental.pallas import tpu as pltpu
from jax.experimental.pallas import tpu_sc as plsc

_SC_NC = 2
_SC_NS = 16
_SC_NW = _SC_NC * _SC_NS


def _dot(a, b, dims):
    return jax.lax.dot_general(a, b, (dims, ((), ())),
                               preferred_element_type=jnp.float32)


def _sc_gather_call(T, idxf):
    NIDX = idxf.shape[0]
    D = T.shape[1]
    per_w = NIDX // _SC_NW
    CH = 96
    n_pair = per_w // (2 * CH)
    mesh = plsc.VectorSubcoreMesh(core_axis_name="c", subcore_axis_name="s")

    @functools.partial(
        pl.kernel, mesh=mesh,
        out_type=jax.ShapeDtypeStruct((NIDX, D), jnp.float32),
        scratch_types=[
            pltpu.VMEM((per_w,), jnp.int32),
            pltpu.VMEM((CH, D), jnp.float32),
            pltpu.VMEM((CH, D), jnp.float32),
            pltpu.SemaphoreType.DMA,
            pltpu.SemaphoreType.DMA,
            pltpu.SemaphoreType.DMA,
            pltpu.SemaphoreType.DMA,
        ],
    )
    def k(table_hbm, idx_hbm, out_hbm, idx_v, rows0, rows1,
          g0, g1, o0, o1):
        wid = jax.lax.axis_index("s") * _SC_NC + jax.lax.axis_index("c")
        base = wid * per_w
        pltpu.sync_copy(idx_hbm.at[pl.ds(base, per_w)], idx_v)

        def gather(c, buf, sem):
            return pltpu.make_async_copy(
                table_hbm.at[idx_v.at[pl.ds(c * CH, CH)]], buf, sem)

        def store(c, buf, sem):
            return pltpu.make_async_copy(
                buf, out_hbm.at[pl.ds(base + c * CH, CH)], sem)

        gather(0, rows0, g0).start()

        @pl.loop(0, n_pair)
        def _(p):
            a = 2 * p
            gather(a, rows0, g0).wait()
            gather(a + 1, rows1, g1).start()
            store(a, rows0, o0).start()
            gather(a + 1, rows1, g1).wait()
            store(a, rows0, o0).wait()

            @pl.when(p + 1 < n_pair)
            def _():
                gather(a + 2, rows0, g0).start()
            store(a + 1, rows1, o1).start()
            store(a + 1, rows1, o1).wait()

    return k(T, idxf)


def _topk_kernel(S, mode, xyz1_ref, xyz2t_ref, idx_ref, w_ref):
    x = xyz1_ref[0]
    q = xyz2t_ref[0]
    nb = x.shape[0]
    cross = jax.lax.dot_general(x.astype(jnp.bfloat16), q.astype(jnp.bfloat16),
                                (((1,), (0,)), ((), ())),
                                preferred_element_type=jnp.float32)
    xs = (x[:, 0:1] * x[:, 0:1] + x[:, 1:2] * x[:, 1:2]) + x[:, 2:3] * x[:, 2:3]
    qs = (q[0:1, :] * q[0:1, :] + q[1:2, :] * q[1:2, :]) + q[2:3, :] * q[2:3, :]
    d = (-2.0 * cross + xs) + qs
    iota = jax.lax.broadcasted_iota(jnp.int32, (nb, S), 1)
    idxs, vals = [], []
    for k in range(3):
        mval = jnp.min(d, axis=1, keepdims=True)
        am = jnp.min(jnp.where(d == mval, iota, S), axis=1, keepdims=True)
        idxs.append(am)
        vals.append(mval)
        if k < 2:
            d = jnp.where(iota == am, jnp.float32(jnp.inf), d)
    d3 = jnp.concatenate(vals, axis=1)
    recip = 1.0 / (d3 + 1e-8)
    w = recip / jnp.sum(recip, axis=1, keepdims=True)
    idx = jnp.concatenate(idxs, axis=1)
    if mode == "sc":
        idx_ref[...] = (idx + pl.program_id(0) * S).T
    else:
        idx_ref[...] = idx
    w_ref[...] = w


def _table_kernel(p2_ref, Wc2_ref, T_ref):
    T_ref[...] = _dot(p2_ref[0], Wc2_ref[...], (((0,), (1,))))


def _fuse_kernel(S, Bh, g0_ref, g1_ref, g2_ref, wa_ref, idxb_ref, wb_ref,
                 p1_ref, T_ref, Wc1_ref, s1_ref, st_ref):
    b = pl.program_id(0)
    j = pl.program_id(1)
    p1dot = _dot(p1_ref[0].astype(jnp.bfloat16),
                 Wc1_ref[...].astype(jnp.bfloat16), (((0,), (1,))))
    nb = p1dot.shape[0]

    @pl.when(b < Bh)
    def _():
        wa = wa_ref[...]
        s1 = (g0_ref[...] * wa[:, 0:1] + g1_ref[...] * wa[:, 1:2]
              + g2_ref[...] * wa[:, 2:3]) + p1dot
        s1_ref[...] = s1.astype(s1_ref.dtype)

    @pl.when(b >= Bh)
    def _():
        idx = idxb_ref[...]
        wb = wb_ref[...]
        iota = jax.lax.broadcasted_iota(jnp.int32, (nb, S), 1)
        oh = jnp.zeros((nb, S), jnp.float32)
        for k in range(3):
            oh = oh + jnp.where(iota == idx[:, k:k + 1], wb[:, k:k + 1], 0.0)
        s1 = _dot(oh, T_ref[...], (((1,), (0,)))) + p1dot
        s1_ref[...] = s1.astype(s1_ref.dtype)

    @pl.when((b == 0) & (j == 0))
    def _():
        st_ref[...] = jnp.zeros_like(st_ref)
    sv = s1_ref[...].astype(jnp.float32)
    st_ref[0:1, :] += jnp.sum(sv, axis=0, keepdims=True)
    st_ref[1:2, :] += jnp.sum(sv * sv, axis=0, keepdims=True)


def _mid_kernel(store_x, s_ref, sc_ref, sh_ref, W_ref, *out_refs):
    if store_x:
        x_ref, s2_ref, st_ref = out_refs
    else:
        s2_ref, st_ref = out_refs
    x = jnp.maximum(s_ref[...].astype(jnp.float32) * sc_ref[...]
                    + sh_ref[...], 0.0)
    xb = x.astype(jnp.bfloat16)
    s2 = _dot(xb, W_ref[...].astype(jnp.bfloat16), (((1,), (1,))))
    if store_x:
        x_ref[...] = xb
    s2b = s2.astype(jnp.bfloat16)
    s2_ref[...] = s2b

    @pl.when(pl.program_id(0) == 0)
    def _():
        st_ref[...] = jnp.zeros_like(st_ref)
    s2f = s2b.astype(jnp.float32)
    st_ref[0:1, :] += jnp.sum(s2f, axis=0, keepdims=True)
    st_ref[1:2, :] += jnp.sum(s2f * s2f, axis=0, keepdims=True)


def _final_kernel(s3_ref, x_ref, sc_ref, sh_ref, o_ref):
    y = (s3_ref[...].astype(jnp.float32) * sc_ref[...] + sh_ref[...]
         + x_ref[...].astype(jnp.float32))
    o_ref[0] = jnp.maximum(y, 0.0).T


def _stats_to_scale_shift(st, nt, g, be, eps):
    mean = st[0] / nt
    var = st[1] / nt - mean * mean
    scale = g / jnp.sqrt(var + eps)
    shift = be - mean * scale
    return scale[None, :], shift[None, :]


def kernel(xyz1, xyz2, points1, points2, W_fuse, b_fuse, g_fuse, be_fuse,
           W1, b1, g1, be1, W2, b2, g2, be2):
    B, N, _ = xyz1.shape
    S = xyz2.shape[1]
    C1 = points1.shape[1]
    C2 = points2.shape[1]
    CO = W_fuse.shape[0]
    NT = B * N
    nb = 1024
    NB = N // nb
    f32 = jnp.float32

    xyz2t = jnp.transpose(xyz2, (0, 2, 1))
    Wc1 = W_fuse[:, :C1]
    Wc2 = W_fuse[:, C1:]
    bf16 = jnp.bfloat16

    Bh = B // 2
    NTh = Bh * N
    NBh = NTh // nb

    T = pl.pallas_call(
        _table_kernel,
        grid=(B,),
        in_specs=[
            pl.BlockSpec((1, C2, S), lambda b: (b, 0, 0)),
            pl.BlockSpec((CO, C2), lambda b: (0, 0)),
        ],
        out_specs=pl.BlockSpec((S, CO), lambda b: (b, 0)),
        out_shape=jax.ShapeDtypeStruct((B * S, CO), f32),
    )(points2, Wc2)

    idxA, wA = pl.pallas_call(
        functools.partial(_topk_kernel, S, "sc"),
        grid=(Bh, NB),
        in_specs=[
            pl.BlockSpec((1, nb, 3), lambda b, j: (b, j, 0)),
            pl.BlockSpec((1, 3, S), lambda b, j: (b, 0, 0)),
        ],
        out_specs=[
            pl.BlockSpec((3, nb), lambda b, j: (0, b * NB + j)),
            pl.BlockSpec((nb, 3), lambda b, j: (b * NB + j, 0)),
        ],
        out_shape=[
            jax.ShapeDtypeStruct((3, NTh), jnp.int32),
            jax.ShapeDtypeStruct((NTh, 3), f32),
        ],
    )(xyz1, xyz2t)

    G = _sc_gather_call(T, jnp.reshape(idxA, (3 * NTh,)))

    idxB, wB = pl.pallas_call(
        functools.partial(_topk_kernel, S, "oh"),
        grid=(Bh, NB),
        in_specs=[
            pl.BlockSpec((1, nb, 3), lambda b, j: (b + Bh, j, 0)),
            pl.BlockSpec((1, 3, S), lambda b, j: (b + Bh, 0, 0)),
        ],
        out_specs=[
            pl.BlockSpec((nb, 3), lambda b, j: (b * NB + j, 0)),
            pl.BlockSpec((nb, 3), lambda b, j: (b * NB + j, 0)),
        ],
        out_shape=[
            jax.ShapeDtypeStruct((NTh, 3), jnp.int32),
            jax.ShapeDtypeStruct((NTh, 3), f32),
        ],
    )(xyz1, xyz2t)

    NBT = NT // nb
    s1, st1 = pl.pallas_call(
        functools.partial(_fuse_kernel, S, Bh),
        grid=(B, NB),
        in_specs=[
            pl.BlockSpec(
                (nb, CO),
                lambda b, j: (0 * NBh + jnp.minimum(b * NB + j, NBh - 1), 0)),
            pl.BlockSpec(
                (nb, CO),
                lambda b, j: (1 * NBh + jnp.minimum(b * NB + j, NBh - 1), 0)),
            pl.BlockSpec(
                (nb, CO),
                lambda b, j: (2 * NBh + jnp.minimum(b * NB + j, NBh - 1), 0)),
            pl.BlockSpec(
                (nb, 3), lambda b, j: (jnp.minimum(b * NB + j, NBh - 1), 0)),
            pl.BlockSpec(
                (nb, 3),
                lambda b, j: (jnp.maximum(b * NB + j - NBh, 0), 0)),
            pl.BlockSpec(
                (nb, 3),
                lambda b, j: (jnp.maximum(b * NB + j - NBh, 0), 0)),
            pl.BlockSpec((1, C1, nb), lambda b, j: (b, 0, j)),
            pl.BlockSpec((S, CO), lambda b, j: (jnp.maximum(b, Bh), 0)),
            pl.BlockSpec((CO, C1), lambda b, j: (0, 0)),
        ],
        out_specs=[
            pl.BlockSpec((nb, CO), lambda b, j: (b * NB + j, 0)),
            pl.BlockSpec((8, CO), lambda b, j: (0, 0)),
        ],
        out_shape=[
            jax.ShapeDtypeStruct((NT, CO), bf16),
            jax.ShapeDtypeStruct((8, CO), f32),
        ],
    )(G, G, G, wA, idxB, wB, points1, T, Wc1)

    sc1, sh1 = _stats_to_scale_shift(st1, NT, g_fuse, be_fuse, 1e-5)

    x, s2, st2 = pl.pallas_call(
        functools.partial(_mid_kernel, True),
        grid=(NBT,),
        in_specs=[
            pl.BlockSpec((nb, CO), lambda i: (i, 0)),
            pl.BlockSpec((1, CO), lambda i: (0, 0)),
            pl.BlockSpec((1, CO), lambda i: (0, 0)),
            pl.BlockSpec((CO, CO), lambda i: (0, 0)),
        ],
        out_specs=[
            pl.BlockSpec((nb, CO), lambda i: (i, 0)),
            pl.BlockSpec((nb, CO), lambda i: (i, 0)),
            pl.BlockSpec((8, CO), lambda i: (0, 0)),
        ],
        out_shape=[
            jax.ShapeDtypeStruct((NT, CO), bf16),
            jax.ShapeDtypeStruct((NT, CO), bf16),
            jax.ShapeDtypeStruct((8, CO), f32),
        ],
    )(s1, sc1, sh1, W1)

    sc2, sh2 = _stats_to_scale_shift(st2, NT, g1, be1, 1e-5)

    s3, st3 = pl.pallas_call(
        functools.partial(_mid_kernel, False),
        grid=(NBT,),
        in_specs=[
            pl.BlockSpec((nb, CO), lambda i: (i, 0)),
            pl.BlockSpec((1, CO), lambda i: (0, 0)),
            pl.BlockSpec((1, CO), lambda i: (0, 0)),
            pl.BlockSpec((CO, CO), lambda i: (0, 0)),
        ],
        out_specs=[
            pl.BlockSpec((nb, CO), lambda i: (i, 0)),
            pl.BlockSpec((8, CO), lambda i: (0, 0)),
        ],
        out_shape=[
            jax.ShapeDtypeStruct((NT, CO), bf16),
            jax.ShapeDtypeStruct((8, CO), f32),
        ],
    )(s2, sc2, sh2, W2)

    sc3, sh3 = _stats_to_scale_shift(st3, NT, g2, be2, 1e-5)

    out = pl.pallas_call(
        _final_kernel,
        grid=(B, NB),
        in_specs=[
            pl.BlockSpec((nb, CO), lambda b, j: (b * NB + j, 0)),
            pl.BlockSpec((nb, CO), lambda b, j: (b * NB + j, 0)),
            pl.BlockSpec((1, CO), lambda b, j: (0, 0)),
            pl.BlockSpec((1, CO), lambda b, j: (0, 0)),
        ],
        out_specs=pl.BlockSpec((1, CO, nb), lambda b, j: (b, 0, j)),
        out_shape=jax.ShapeDtypeStruct((B, CO, N), f32),
    )(s3, x, sc3, sh3)

    return out
